# Initial kernel scaffold; baseline (speedup 1.0000x reference)
import jax
import jax.numpy as jnp
from jax.experimental import pallas as pl


def kernel(x, edge_index, edge_attr, tupleid, tuplefeat, tuple_edge_index, tuple_edge_base, batch, W_tupleinit, b_tupleinit, conv_W, conv_b, W_merge, b_merge, W_pred, b_pred):
    raise NotImplementedError("write your pallas kernel here")


# speedup vs baseline: 1.7984x; 1.7984x over previous
import functools

import jax
import jax.numpy as jnp
from jax import lax
from jax.experimental import pallas as pl
from jax.experimental.pallas import tpu as pltpu
from jax.experimental.pallas import tpu_sc as plsc

N = 10000
E = 160000
S = 8
T = N * S
TE = 320000
D = 256
NUM_TASKS = 10
NUM_GRAPHS = 64

NC = 2
NS = 16

NP = 8
RP = T // NP
R_ACC = RP + 240

LPT = 5120
NCH_L = 40
R_ACC2 = N + 240
ZR = R_ACC2 // NS

GPW = 2560
NCH_G = GPW // 128

_mesh = plsc.VectorSubcoreMesh(core_axis_name="c", subcore_axis_name="s")


@functools.partial(
    pl.kernel,
    out_type=jax.ShapeDtypeStruct((T, D), jnp.float32),
    mesh=_mesh,
    scratch_types=[
        pltpu.VMEM((NCH_G, 128), jnp.int32),
        pltpu.VMEM((128, D), jnp.float32),
        pltpu.SemaphoreType.DMA,
    ],
)
def sc_gather_rows(table_hbm, idx_hbm, out_hbm, idx_v, buf, sem):
    cid = lax.axis_index("c")
    sid = lax.axis_index("s")
    wid = sid * NC + cid
    pltpu.sync_copy(idx_hbm.at[wid], idx_v)
    nch = jnp.where(wid == NC * NS - 1, 5, NCH_G)

    def body(j, _):
        pltpu.async_copy(table_hbm.at[idx_v.at[j]], buf, sem).wait()
        pltpu.sync_copy(buf, out_hbm.at[pl.ds(wid * GPW + j * 128, 128)])
        return 0

    lax.fori_loop(0, nch, body, 0)


@functools.partial(
    pl.kernel,
    out_type=jax.ShapeDtypeStruct((T, D), jnp.float32),
    mesh=_mesh,
    scratch_types=[
        pltpu.VMEM_SHARED((R_ACC, 128), jnp.float32),
        pltpu.VMEM((16,), jnp.int32),
        pltpu.VMEM((128,), jnp.int32),
        pltpu.VMEM((128,), jnp.int32),
        pltpu.VMEM((128,), jnp.int32),
        pltpu.VMEM((128,), jnp.int32),
        pltpu.VMEM((128, 128), jnp.float32),
        pltpu.VMEM((128, 128), jnp.float32),
        pltpu.VMEM((80, 128), jnp.float32),
        pltpu.SemaphoreType.DMA,
        pltpu.SemaphoreType.DMA,
    ],
)
def sc_layer(x_hbm, ea_hbm, src_hbm, base_hbm, dst_hbm, meta_hbm, agg_hbm,
             acc, meta_v, src_v, base_v, dst_v, dstl_v, bufx, bufe,
             stage, sem1, sem2):
    cid = lax.axis_index("c")
    sid = lax.axis_index("s")
    c128 = cid * 128
    xh = x_hbm.at[cid]
    eah = ea_hbm.at[cid]
    pltpu.sync_copy(meta_hbm, meta_v)
    mv = meta_v[...]

    for p in range(NP):
        r0 = p * RP
        sc0 = mv[p]
        ncp = mv[NP + p]

        def zf(i, _):
            stage[i // 8, pl.ds((i % 8) * 16, 16)] = jnp.zeros((16,),
                                                               jnp.float32)
            return 0
        lax.fori_loop(0, 80 * 8, zf, 0)
        for q in range(8):
            pltpu.sync_copy(stage, acc.at[pl.ds(sid * 640 + q * 80, 80)])
        plsc.subcore_barrier()

        nkk = jnp.maximum((ncp - sid + NS - 1) // NS, 0)

        def chunk(kk2, _):
            e0 = (sc0 + sid + kk2 * NS) * 128
            pltpu.sync_copy(src_hbm.at[pl.ds(e0, 128)], src_v)
            pltpu.sync_copy(base_hbm.at[pl.ds(e0, 128)], base_v)
            pltpu.sync_copy(dst_hbm.at[pl.ds(e0, 128)], dst_v)
            g1 = pltpu.async_copy(xh.at[src_v], bufx, sem1)
            g2 = pltpu.async_copy(eah.at[base_v], bufe, sem2)
            for g in range(8):
                d = dst_v[pl.ds(g * 16, 16)] - r0
                d = jnp.where((d >= 0) & (d < RP), d, RP)
                dstl_v[pl.ds(g * 16, 16)] = d
            g1.wait()
            g2.wait()

            def relu_add(i, _):
                bufx[i] = jnp.maximum(bufx[i] + bufe[i], 0.0)
                return 0

            lax.fori_loop(0, 128, relu_add, 0)
            pltpu.sync_copy(bufx, acc.at[dstl_v], add=True)
            return 0

        lax.fori_loop(0, nkk, chunk, 0)
        plsc.subcore_barrier()
        a0 = sid * 624
        for q in range(7):
            pltpu.sync_copy(acc.at[pl.ds(a0 + q * 80, 80)], stage)
            pltpu.sync_copy(stage,
                            agg_hbm.at[pl.ds(r0 + a0 + q * 80, 80),
                                       pl.ds(c128, 128)])

        @pl.when(sid < NS - 1)
        def _():
            pltpu.sync_copy(acc.at[pl.ds(a0 + 560, 64)],
                            stage.at[pl.ds(0, 64)])
            pltpu.sync_copy(stage.at[pl.ds(0, 64)],
                            agg_hbm.at[pl.ds(r0 + a0 + 560, 64),
                                       pl.ds(c128, 128)])

        @pl.when(sid == NS - 1)
        def _():
            pltpu.sync_copy(acc.at[pl.ds(a0 + 560, 80)], stage)
            pltpu.sync_copy(stage,
                            agg_hbm.at[pl.ds(r0 + a0 + 560, 80),
                                       pl.ds(c128, 128)])

        plsc.subcore_barrier()


@functools.partial(
    pl.kernel,
    out_type=jax.ShapeDtypeStruct((N, D), jnp.float32),
    mesh=_mesh,
    scratch_types=[
        pltpu.VMEM_SHARED((R_ACC2, 128), jnp.float32),
        pltpu.VMEM((128,), jnp.int32),
        pltpu.VMEM((128,), jnp.int32),
        pltpu.VMEM((128,), jnp.int32),
        pltpu.VMEM((128, 128), jnp.float32),
        pltpu.VMEM((80, 128), jnp.float32),
        pltpu.SemaphoreType.DMA,
    ],
)
def sc_pool_leaf(x_hbm, gidx_hbm, sidx_hbm, x2_hbm,
                 acc, gidx_v, sidx_v, sidx2_v, buf, stage, sem):
    cid = lax.axis_index("c")
    sid = lax.axis_index("s")
    c128 = cid * 128
    xh = x_hbm.at[cid]

    def zf(i, _):
        stage[i // 8, pl.ds((i % 8) * 16, 16)] = jnp.zeros((16,), jnp.float32)
        return 0
    lax.fori_loop(0, 80 * 8, zf, 0)
    for q in range(8):
        pltpu.sync_copy(stage, acc.at[pl.ds(sid * 640 + q * 80, 80)])
    plsc.subcore_barrier()

    def chunk(kk, _):
        pltpu.sync_copy(gidx_hbm.at[sid, kk], gidx_v)
        pltpu.sync_copy(sidx_hbm.at[sid, kk], sidx_v)
        g1 = pltpu.async_copy(xh.at[gidx_v], buf, sem)
        for g in range(8):
            sidx2_v[pl.ds(g * 16, 16)] = sidx_v[pl.ds(g * 16, 16)]
        g1.wait()
        pltpu.sync_copy(buf, acc.at[sidx2_v], add=True)
        return 0

    lax.fori_loop(0, NCH_L, chunk, 0)
    plsc.subcore_barrier()

    r0 = sid * 624
    for q in range(7):
        pltpu.sync_copy(acc.at[pl.ds(r0 + q * 80, 80)], stage)
        pltpu.sync_copy(stage,
                        x2_hbm.at[pl.ds(r0 + q * 80, 80), pl.ds(c128, 128)])

    @pl.when(sid < NS - 1)
    def _():
        pltpu.sync_copy(acc.at[pl.ds(r0 + 560, 64)], stage.at[pl.ds(0, 64)])
        pltpu.sync_copy(stage.at[pl.ds(0, 64)],
                        x2_hbm.at[pl.ds(r0 + 560, 64), pl.ds(c128, 128)])

    @pl.when(sid == NS - 1)
    def _():
        pltpu.sync_copy(acc.at[pl.ds(r0 + 560, 80)], stage)
        pltpu.sync_copy(stage,
                        x2_hbm.at[pl.ds(r0 + 560, 80), pl.ds(c128, 128)])


@functools.partial(
    pl.kernel,
    out_type=jax.ShapeDtypeStruct((NS, R_ACC2), jnp.float32),
    mesh=_mesh,
    scratch_types=[
        pltpu.VMEM((128,), jnp.int32),
        pltpu.VMEM((R_ACC2,), jnp.float32),
    ],
    compiler_params=pltpu.CompilerParams(needs_layout_passes=False),
)
def sc_hist(leaf_hbm, hist_hbm, leaf_v, hist):
    cid = lax.axis_index("c")
    sid = lax.axis_index("s")

    @pl.when(cid == 0)
    def _():
        def hz(i, _):
            hist[pl.ds(i * 16, 16)] = jnp.zeros((16,), jnp.float32)
            return 0
        lax.fori_loop(0, R_ACC2 // 16, hz, 0)

        ones16 = jnp.ones((16,), jnp.float32)

        def chunk(kk, _):
            pltpu.sync_copy(leaf_hbm.at[sid, kk], leaf_v)
            for g in range(8):
                idx = leaf_v[pl.ds(g * 16, 16)]
                plsc.addupdate_scatter(hist, [idx], ones16)
            return 0

        lax.fori_loop(0, NCH_L, chunk, 0)
        pltpu.sync_copy(hist, hist_hbm.at[sid])


def _tc_xt_body(x_ref, w_ref, b_ref, o_ref):
    o_ref[...] = jnp.dot(x_ref[...], w_ref[...],
                         preferred_element_type=jnp.float32) + b_ref[...]


def tc_xt(x, w, b):
    return pl.pallas_call(
        _tc_xt_body,
        grid=(10,),
        in_specs=[
            pl.BlockSpec((1000, D), lambda i: (i, 0)),
            pl.BlockSpec((D, D), lambda i: (0, 0)),
            pl.BlockSpec((1, D), lambda i: (0, 0)),
        ],
        out_specs=pl.BlockSpec((1000, D), lambda i: (i, 0)),
        out_shape=jax.ShapeDtypeStruct((N, D), jnp.float32),
    )(x, w, b)


def _tc_init_body(x_ref, g_ref, tf_ref, o_ref, o2_ref):
    xr = jnp.repeat(x_ref[...], S, axis=0)
    xn = xr * g_ref[...] * tf_ref[...]
    o_ref[...] = xn
    o2_ref[0] = xn[:, :128]
    o2_ref[1] = xn[:, 128:]


def tc_init(x, g, tf):
    return pl.pallas_call(
        _tc_init_body,
        grid=(125,),
        in_specs=[
            pl.BlockSpec((80, D), lambda i: (i, 0)),
            pl.BlockSpec((640, D), lambda i: (i, 0)),
            pl.BlockSpec((640, D), lambda i: (i, 0)),
        ],
        out_specs=[pl.BlockSpec((640, D), lambda i: (i, 0)),
                   pl.BlockSpec((2, 640, 128), lambda i: (0, i, 0))],
        out_shape=(jax.ShapeDtypeStruct((T, D), jnp.float32),
                   jax.ShapeDtypeStruct((2, T, 128), jnp.float32)),
    )(x, g, tf)


def _tc_conv_body(x_ref, a_ref, w_ref, b_ref, o_ref, o2_ref):
    up = jnp.dot(a_ref[...], w_ref[...],
                 preferred_element_type=jnp.float32) + b_ref[...]
    xn = x_ref[...] + jnp.maximum(up, 0.0)
    o_ref[...] = xn
    o2_ref[0] = xn[:, :128]
    o2_ref[1] = xn[:, 128:]


def tc_conv(x, agg, w, b):
    return pl.pallas_call(
        _tc_conv_body,
        grid=(125,),
        in_specs=[
            pl.BlockSpec((640, D), lambda i: (i, 0)),
            pl.BlockSpec((640, D), lambda i: (i, 0)),
            pl.BlockSpec((D, D), lambda i: (0, 0)),
            pl.BlockSpec((1, D), lambda i: (0, 0)),
        ],
        out_specs=[pl.BlockSpec((640, D), lambda i: (i, 0)),
                   pl.BlockSpec((2, 640, 128), lambda i: (0, i, 0))],
        out_shape=(jax.ShapeDtypeStruct((T, D), jnp.float32),
                   jax.ShapeDtypeStruct((2, T, 128), jnp.float32)),
    )(x, agg, w, b)


def _tc_split_body(x_ref, o_ref):
    o_ref[0] = x_ref[:, :128]
    o_ref[1] = x_ref[:, 128:]


def tc_split(x):
    n = x.shape[0]
    return pl.pallas_call(
        _tc_split_body,
        grid=(n // 640,),
        in_specs=[pl.BlockSpec((640, D), lambda i: (i, 0))],
        out_specs=pl.BlockSpec((2, 640, 128), lambda i: (0, i, 0)),
        out_shape=jax.ShapeDtypeStruct((2, n, 128), jnp.float32),
    )(x)


NB = 400
NBLK = N // NB


def _tc_final_body(x_ref, x2_ref, hist_ref, batch_ref, wm_ref, bm_ref,
                   wp_ref, bp_ref, o_ref, hg_acc, cnt_acc):
    i = pl.program_id(0)

    @pl.when(i == 0)
    def _():
        hg_acc[...] = jnp.zeros_like(hg_acc)
        cnt_acc[...] = jnp.zeros_like(cnt_acc)

    xb = x_ref[...].reshape(NB, S, D)
    x1 = jnp.sum(xb, axis=1) * (1.0 / S)
    x3 = xb[:, 0, :]
    histsum = jnp.sum(hist_ref[0], axis=0)[:, None]
    x2 = x2_ref[...] / jnp.maximum(histsum, 1.0)
    h = (jnp.dot(x1, wm_ref[0:D], preferred_element_type=jnp.float32)
         + jnp.dot(x2, wm_ref[D:2 * D], preferred_element_type=jnp.float32)
         + jnp.dot(x3, wm_ref[2 * D:3 * D], preferred_element_type=jnp.float32)
         + bm_ref[...])
    h = jnp.maximum(h, 0.0)
    gids = lax.broadcasted_iota(jnp.int32, (NUM_GRAPHS, NB), 0)
    m = (gids == batch_ref[0]).astype(jnp.float32)
    hg_acc[...] += jnp.dot(m, h, preferred_element_type=jnp.float32)
    cnt = jnp.sum(m, axis=1, keepdims=True)
    cnt_acc[...] += jnp.broadcast_to(cnt, cnt_acc.shape)

    @pl.when(i == NBLK - 1)
    def _():
        hg = hg_acc[...] / jnp.maximum(cnt_acc[:, 0:1], 1.0)
        o_ref[...] = jnp.dot(hg, wp_ref[...],
                             preferred_element_type=jnp.float32) + bp_ref[...]


def tc_final(x, x2sum, hist_r, batch_r, wm, bm, wp, bp):
    return pl.pallas_call(
        _tc_final_body,
        grid=(NBLK,),
        in_specs=[
            pl.BlockSpec((NB * S, D), lambda i: (i, 0)),
            pl.BlockSpec((NB, D), lambda i: (i, 0)),
            pl.BlockSpec((1, NS, NB), lambda i: (i, 0, 0)),
            pl.BlockSpec((1, 1, NB), lambda i: (i, 0, 0)),
            pl.BlockSpec((3 * D, D), lambda i: (0, 0)),
            pl.BlockSpec((1, D), lambda i: (0, 0)),
            pl.BlockSpec((D, NUM_TASKS), lambda i: (0, 0)),
            pl.BlockSpec((1, NUM_TASKS), lambda i: (0, 0)),
        ],
        out_specs=pl.BlockSpec((NUM_GRAPHS, NUM_TASKS), lambda i: (0, 0)),
        out_shape=jax.ShapeDtypeStruct((NUM_GRAPHS, NUM_TASKS), jnp.float32),
        scratch_shapes=[
            pltpu.VMEM((NUM_GRAPHS, D), jnp.float32),
            pltpu.VMEM((NUM_GRAPHS, 128), jnp.float32),
        ],
    )(x, x2sum, hist_r, batch_r, wm, bm, wp, bp)


def _pad_reshape(a, total, fill, shape):
    pad = total - a.shape[0]
    return jnp.concatenate(
        [a, jnp.full((pad,), fill, a.dtype)]).reshape(shape)


def kernel(x, edge_index, edge_attr, tupleid, tuplefeat, tuple_edge_index,
           tuple_edge_base, batch, W_tupleinit, b_tupleinit, conv_W, conv_b,
           W_merge, b_merge, W_pred, b_pred):
    leaf = tupleid[1]
    src = tuple_edge_index[0]
    dst = tuple_edge_index[1]

    leaf_g = _pad_reshape(leaf, NC * NS * GPW, 0, (NC * NS, NCH_G, 128))
    leaf_s = _pad_reshape(leaf, NS * LPT, N, (NS, NCH_L, 128))
    lorder = jnp.argsort(leaf).astype(jnp.int32)
    lorder_s = _pad_reshape(lorder, NS * LPT, 0, (NS, NCH_L, 128))
    leafsort_s = _pad_reshape(leaf[lorder], NS * LPT, N, (NS, NCH_L, 128))

    order = jnp.argsort(dst)
    src_s = src[order]
    base_s = tuple_edge_base[order]
    dst_s = dst[order]
    bounds = jnp.searchsorted(dst_s, jnp.arange(0, T + 1, RP, dtype=jnp.int32))
    start_c = (bounds[:-1] // 128).astype(jnp.int32)
    end_c = ((bounds[1:] + 127) // 128).astype(jnp.int32)
    meta = jnp.concatenate([start_c, end_c - start_c]).astype(jnp.int32)

    b_ti = b_tupleinit.reshape(1, D)
    b_m = b_merge.reshape(1, D)
    b_p = b_pred.reshape(1, NUM_TASKS)

    xt = tc_xt(x, W_tupleinit, b_ti)
    G = sc_gather_rows(xt, leaf_g)
    X, X2 = tc_init(x, G, tuplefeat)
    ea2 = tc_split(edge_attr)
    for l in range(conv_W.shape[0]):
        agg = sc_layer(X2, ea2, src_s, base_s, dst_s, meta)
        X, X2 = tc_conv(X, agg, conv_W[l], conv_b[l].reshape(1, D))
    x2sum = sc_pool_leaf(X2, lorder_s, leafsort_s)
    hist = sc_hist(leaf_s)

    hist_r = hist[:, :N].reshape(NS, NBLK, NB).transpose(1, 0, 2)
    batch_r = batch.reshape(NBLK, 1, NB)
    out = tc_final(X, x2sum, hist_r, batch_r, W_merge, b_m, W_pred, b_p)
    return out


# speedup vs baseline: 1.9691x; 1.0949x over previous
import functools

import jax
import jax.numpy as jnp
from jax import lax
from jax.experimental import pallas as pl
from jax.experimental.pallas import tpu as pltpu
from jax.experimental.pallas import tpu_sc as plsc

N = 10000
E = 160000
S = 8
T = N * S
TE = 320000
D = 256
NUM_TASKS = 10
NUM_GRAPHS = 64

NC = 2
NS = 16

NP = 8
RP = T // NP
R_ACC = RP + 240

LPT = 5120
NCH_L = 40
R_ACC2 = N + 240
ZR = R_ACC2 // NS

GPW = 2560
NCH_G = GPW // 128

_mesh = plsc.VectorSubcoreMesh(core_axis_name="c", subcore_axis_name="s")


@functools.partial(
    pl.kernel,
    out_type=jax.ShapeDtypeStruct((T, D), jnp.float32),
    mesh=_mesh,
    scratch_types=[
        pltpu.VMEM((NCH_G, 128), jnp.int32),
        pltpu.VMEM((128, D), jnp.float32),
        pltpu.SemaphoreType.DMA,
    ],
)
def sc_gather_rows(table_hbm, idx_hbm, out_hbm, idx_v, buf, sem):
    cid = lax.axis_index("c")
    sid = lax.axis_index("s")
    wid = sid * NC + cid
    pltpu.sync_copy(idx_hbm.at[wid], idx_v)
    nch = jnp.where(wid == NC * NS - 1, 5, NCH_G)

    def body(j, _):
        pltpu.async_copy(table_hbm.at[idx_v.at[j]], buf, sem).wait()
        pltpu.sync_copy(buf, out_hbm.at[pl.ds(wid * GPW + j * 128, 128)])
        return 0

    lax.fori_loop(0, nch, body, 0)


@functools.partial(
    pl.kernel,
    out_type=jax.ShapeDtypeStruct((T, D), jnp.float32),
    mesh=_mesh,
    scratch_types=[
        pltpu.VMEM_SHARED((R_ACC, 128), jnp.float32),
        pltpu.VMEM((16,), jnp.int32),
        pltpu.VMEM((128,), jnp.int32),
        pltpu.VMEM((128,), jnp.int32),
        pltpu.VMEM((128,), jnp.int32),
        pltpu.VMEM((128,), jnp.int32),
        pltpu.VMEM((128, 128), jnp.float32),
        pltpu.VMEM((128, 128), jnp.float32),
        pltpu.VMEM((80, 128), jnp.float32),
        pltpu.SemaphoreType.DMA,
        pltpu.SemaphoreType.DMA,
        pltpu.SemaphoreType.DMA,
        pltpu.SemaphoreType.DMA,
        pltpu.SemaphoreType.DMA,
    ],
)
def sc_layer(x_hbm, ea_hbm, src_hbm, base_hbm, dst_hbm, meta_hbm, agg_hbm,
             acc, meta_v, src_v, base_v, dst_v, dstl_v, bufx, bufe,
             stage, sem1, sem2, sem3, sem4, sem5):
    cid = lax.axis_index("c")
    sid = lax.axis_index("s")
    c128 = cid * 128
    xh = x_hbm.at[cid]
    eah = ea_hbm.at[cid]
    pltpu.sync_copy(meta_hbm, meta_v)
    mv = meta_v[...]

    for p in range(NP):
        r0 = p * RP
        sc0 = mv[p]
        ncp = mv[NP + p]

        def zf(i, _):
            stage[i // 8, pl.ds((i % 8) * 16, 16)] = jnp.zeros((16,),
                                                               jnp.float32)
            return 0
        lax.fori_loop(0, 80 * 8, zf, 0)
        for q in range(8):
            pltpu.sync_copy(stage, acc.at[pl.ds(sid * 640 + q * 80, 80)])
        plsc.subcore_barrier()

        nkk = jnp.maximum((ncp - sid + NS - 1) // NS, 0)

        def chunk(kk2, _):
            e0 = (sc0 + sid + kk2 * NS) * 128
            i1 = pltpu.async_copy(src_hbm.at[pl.ds(e0, 128)], src_v, sem3)
            i2 = pltpu.async_copy(base_hbm.at[pl.ds(e0, 128)], base_v, sem4)
            i3 = pltpu.async_copy(dst_hbm.at[pl.ds(e0, 128)], dst_v, sem5)
            i1.wait()
            g1 = pltpu.async_copy(xh.at[src_v], bufx, sem1)
            i2.wait()
            g2 = pltpu.async_copy(eah.at[base_v], bufe, sem2)
            i3.wait()
            for g in range(8):
                d = dst_v[pl.ds(g * 16, 16)] - r0
                d = jnp.where((d >= 0) & (d < RP), d, RP)
                dstl_v[pl.ds(g * 16, 16)] = d
            g1.wait()
            g2.wait()

            def relu_add(i, _):
                bufx[i] = jnp.maximum(bufx[i] + bufe[i], 0.0)
                return 0

            lax.fori_loop(0, 128, relu_add, 0)
            pltpu.sync_copy(bufx, acc.at[dstl_v], add=True)
            return 0

        lax.fori_loop(0, nkk, chunk, 0)
        plsc.subcore_barrier()
        a0 = sid * 624
        for q in range(7):
            pltpu.sync_copy(acc.at[pl.ds(a0 + q * 80, 80)], stage)
            pltpu.sync_copy(stage,
                            agg_hbm.at[pl.ds(r0 + a0 + q * 80, 80),
                                       pl.ds(c128, 128)])

        @pl.when(sid < NS - 1)
        def _():
            pltpu.sync_copy(acc.at[pl.ds(a0 + 560, 64)],
                            stage.at[pl.ds(0, 64)])
            pltpu.sync_copy(stage.at[pl.ds(0, 64)],
                            agg_hbm.at[pl.ds(r0 + a0 + 560, 64),
                                       pl.ds(c128, 128)])

        @pl.when(sid == NS - 1)
        def _():
            pltpu.sync_copy(acc.at[pl.ds(a0 + 560, 80)], stage)
            pltpu.sync_copy(stage,
                            agg_hbm.at[pl.ds(r0 + a0 + 560, 80),
                                       pl.ds(c128, 128)])

        plsc.subcore_barrier()


@functools.partial(
    pl.kernel,
    out_type=jax.ShapeDtypeStruct((N, D), jnp.float32),
    mesh=_mesh,
    scratch_types=[
        pltpu.VMEM_SHARED((R_ACC2, 128), jnp.float32),
        pltpu.VMEM((128,), jnp.int32),
        pltpu.VMEM((128,), jnp.int32),
        pltpu.VMEM((128,), jnp.int32),
        pltpu.VMEM((128, 128), jnp.float32),
        pltpu.VMEM((80, 128), jnp.float32),
        pltpu.SemaphoreType.DMA,
    ],
)
def sc_pool_leaf(x_hbm, gidx_hbm, sidx_hbm, x2_hbm,
                 acc, gidx_v, sidx_v, sidx2_v, buf, stage, sem):
    cid = lax.axis_index("c")
    sid = lax.axis_index("s")
    c128 = cid * 128
    xh = x_hbm.at[cid]

    def zf(i, _):
        stage[i // 8, pl.ds((i % 8) * 16, 16)] = jnp.zeros((16,), jnp.float32)
        return 0
    lax.fori_loop(0, 80 * 8, zf, 0)
    for q in range(8):
        pltpu.sync_copy(stage, acc.at[pl.ds(sid * 640 + q * 80, 80)])
    plsc.subcore_barrier()

    def chunk(kk, _):
        pltpu.sync_copy(gidx_hbm.at[sid, kk], gidx_v)
        pltpu.sync_copy(sidx_hbm.at[sid, kk], sidx_v)
        g1 = pltpu.async_copy(xh.at[gidx_v], buf, sem)
        for g in range(8):
            sidx2_v[pl.ds(g * 16, 16)] = sidx_v[pl.ds(g * 16, 16)]
        g1.wait()
        pltpu.sync_copy(buf, acc.at[sidx2_v], add=True)
        return 0

    lax.fori_loop(0, NCH_L, chunk, 0)
    plsc.subcore_barrier()

    r0 = sid * 624
    for q in range(7):
        pltpu.sync_copy(acc.at[pl.ds(r0 + q * 80, 80)], stage)
        pltpu.sync_copy(stage,
                        x2_hbm.at[pl.ds(r0 + q * 80, 80), pl.ds(c128, 128)])

    @pl.when(sid < NS - 1)
    def _():
        pltpu.sync_copy(acc.at[pl.ds(r0 + 560, 64)], stage.at[pl.ds(0, 64)])
        pltpu.sync_copy(stage.at[pl.ds(0, 64)],
                        x2_hbm.at[pl.ds(r0 + 560, 64), pl.ds(c128, 128)])

    @pl.when(sid == NS - 1)
    def _():
        pltpu.sync_copy(acc.at[pl.ds(r0 + 560, 80)], stage)
        pltpu.sync_copy(stage,
                        x2_hbm.at[pl.ds(r0 + 560, 80), pl.ds(c128, 128)])


@functools.partial(
    pl.kernel,
    out_type=jax.ShapeDtypeStruct((NS, R_ACC2), jnp.float32),
    mesh=_mesh,
    scratch_types=[
        pltpu.VMEM((128,), jnp.int32),
        pltpu.VMEM((R_ACC2,), jnp.float32),
    ],
    compiler_params=pltpu.CompilerParams(needs_layout_passes=False),
)
def sc_hist(leaf_hbm, hist_hbm, leaf_v, hist):
    cid = lax.axis_index("c")
    sid = lax.axis_index("s")

    @pl.when(cid == 0)
    def _():
        def hz(i, _):
            hist[pl.ds(i * 16, 16)] = jnp.zeros((16,), jnp.float32)
            return 0
        lax.fori_loop(0, R_ACC2 // 16, hz, 0)

        ones16 = jnp.ones((16,), jnp.float32)

        def chunk(kk, _):
            pltpu.sync_copy(leaf_hbm.at[sid, kk], leaf_v)
            for g in range(8):
                idx = leaf_v[pl.ds(g * 16, 16)]
                plsc.addupdate_scatter(hist, [idx], ones16)
            return 0

        lax.fori_loop(0, NCH_L, chunk, 0)
        pltpu.sync_copy(hist, hist_hbm.at[sid])


def _tc_xt_body(x_ref, w_ref, b_ref, o_ref):
    o_ref[...] = jnp.dot(x_ref[...], w_ref[...],
                         preferred_element_type=jnp.float32) + b_ref[...]


def tc_xt(x, w, b):
    return pl.pallas_call(
        _tc_xt_body,
        grid=(10,),
        in_specs=[
            pl.BlockSpec((1000, D), lambda i: (i, 0)),
            pl.BlockSpec((D, D), lambda i: (0, 0)),
            pl.BlockSpec((1, D), lambda i: (0, 0)),
        ],
        out_specs=pl.BlockSpec((1000, D), lambda i: (i, 0)),
        out_shape=jax.ShapeDtypeStruct((N, D), jnp.float32),
    )(x, w, b)


def _tc_init_body(x_ref, g_ref, tf_ref, o_ref, o2_ref):
    xr = jnp.repeat(x_ref[...], S, axis=0)
    xn = xr * g_ref[...] * tf_ref[...]
    o_ref[...] = xn
    o2_ref[0] = xn[:, :128]
    o2_ref[1] = xn[:, 128:]


def tc_init(x, g, tf):
    return pl.pallas_call(
        _tc_init_body,
        grid=(125,),
        in_specs=[
            pl.BlockSpec((80, D), lambda i: (i, 0)),
            pl.BlockSpec((640, D), lambda i: (i, 0)),
            pl.BlockSpec((640, D), lambda i: (i, 0)),
        ],
        out_specs=[pl.BlockSpec((640, D), lambda i: (i, 0)),
                   pl.BlockSpec((2, 640, 128), lambda i: (0, i, 0))],
        out_shape=(jax.ShapeDtypeStruct((T, D), jnp.float32),
                   jax.ShapeDtypeStruct((2, T, 128), jnp.float32)),
    )(x, g, tf)


def _tc_conv_body(x_ref, a_ref, w_ref, b_ref, o_ref, o2_ref):
    up = jnp.dot(a_ref[...], w_ref[...],
                 preferred_element_type=jnp.float32) + b_ref[...]
    xn = x_ref[...] + jnp.maximum(up, 0.0)
    o_ref[...] = xn
    o2_ref[0] = xn[:, :128]
    o2_ref[1] = xn[:, 128:]


def tc_conv(x, agg, w, b):
    return pl.pallas_call(
        _tc_conv_body,
        grid=(125,),
        in_specs=[
            pl.BlockSpec((640, D), lambda i: (i, 0)),
            pl.BlockSpec((640, D), lambda i: (i, 0)),
            pl.BlockSpec((D, D), lambda i: (0, 0)),
            pl.BlockSpec((1, D), lambda i: (0, 0)),
        ],
        out_specs=[pl.BlockSpec((640, D), lambda i: (i, 0)),
                   pl.BlockSpec((2, 640, 128), lambda i: (0, i, 0))],
        out_shape=(jax.ShapeDtypeStruct((T, D), jnp.float32),
                   jax.ShapeDtypeStruct((2, T, 128), jnp.float32)),
    )(x, agg, w, b)


def _tc_split_body(x_ref, o_ref):
    o_ref[0] = x_ref[:, :128]
    o_ref[1] = x_ref[:, 128:]


def tc_split(x):
    n = x.shape[0]
    return pl.pallas_call(
        _tc_split_body,
        grid=(n // 640,),
        in_specs=[pl.BlockSpec((640, D), lambda i: (i, 0))],
        out_specs=pl.BlockSpec((2, 640, 128), lambda i: (0, i, 0)),
        out_shape=jax.ShapeDtypeStruct((2, n, 128), jnp.float32),
    )(x)


NB = 400
NBLK = N // NB


def _tc_final_body(x_ref, x2_ref, hist_ref, batch_ref, wm_ref, bm_ref,
                   wp_ref, bp_ref, o_ref, hg_acc, cnt_acc):
    i = pl.program_id(0)

    @pl.when(i == 0)
    def _():
        hg_acc[...] = jnp.zeros_like(hg_acc)
        cnt_acc[...] = jnp.zeros_like(cnt_acc)

    xb = x_ref[...].reshape(NB, S, D)
    x1 = jnp.sum(xb, axis=1) * (1.0 / S)
    x3 = xb[:, 0, :]
    histsum = jnp.sum(hist_ref[0], axis=0)[:, None]
    x2 = x2_ref[...] / jnp.maximum(histsum, 1.0)
    h = (jnp.dot(x1, wm_ref[0:D], preferred_element_type=jnp.float32)
         + jnp.dot(x2, wm_ref[D:2 * D], preferred_element_type=jnp.float32)
         + jnp.dot(x3, wm_ref[2 * D:3 * D], preferred_element_type=jnp.float32)
         + bm_ref[...])
    h = jnp.maximum(h, 0.0)
    gids = lax.broadcasted_iota(jnp.int32, (NUM_GRAPHS, NB), 0)
    m = (gids == batch_ref[0]).astype(jnp.float32)
    hg_acc[...] += jnp.dot(m, h, preferred_element_type=jnp.float32)
    cnt = jnp.sum(m, axis=1, keepdims=True)
    cnt_acc[...] += jnp.broadcast_to(cnt, cnt_acc.shape)

    @pl.when(i == NBLK - 1)
    def _():
        hg = hg_acc[...] / jnp.maximum(cnt_acc[:, 0:1], 1.0)
        o_ref[...] = jnp.dot(hg, wp_ref[...],
                             preferred_element_type=jnp.float32) + bp_ref[...]


def tc_final(x, x2sum, hist_r, batch_r, wm, bm, wp, bp):
    return pl.pallas_call(
        _tc_final_body,
        grid=(NBLK,),
        in_specs=[
            pl.BlockSpec((NB * S, D), lambda i: (i, 0)),
            pl.BlockSpec((NB, D), lambda i: (i, 0)),
            pl.BlockSpec((1, NS, NB), lambda i: (i, 0, 0)),
            pl.BlockSpec((1, 1, NB), lambda i: (i, 0, 0)),
            pl.BlockSpec((3 * D, D), lambda i: (0, 0)),
            pl.BlockSpec((1, D), lambda i: (0, 0)),
            pl.BlockSpec((D, NUM_TASKS), lambda i: (0, 0)),
            pl.BlockSpec((1, NUM_TASKS), lambda i: (0, 0)),
        ],
        out_specs=pl.BlockSpec((NUM_GRAPHS, NUM_TASKS), lambda i: (0, 0)),
        out_shape=jax.ShapeDtypeStruct((NUM_GRAPHS, NUM_TASKS), jnp.float32),
        scratch_shapes=[
            pltpu.VMEM((NUM_GRAPHS, D), jnp.float32),
            pltpu.VMEM((NUM_GRAPHS, 128), jnp.float32),
        ],
    )(x, x2sum, hist_r, batch_r, wm, bm, wp, bp)


def _pad_reshape(a, total, fill, shape):
    pad = total - a.shape[0]
    return jnp.concatenate(
        [a, jnp.full((pad,), fill, a.dtype)]).reshape(shape)


def kernel(x, edge_index, edge_attr, tupleid, tuplefeat, tuple_edge_index,
           tuple_edge_base, batch, W_tupleinit, b_tupleinit, conv_W, conv_b,
           W_merge, b_merge, W_pred, b_pred):
    leaf = tupleid[1]
    src = tuple_edge_index[0]
    dst = tuple_edge_index[1]

    leaf_g = _pad_reshape(leaf, NC * NS * GPW, 0, (NC * NS, NCH_G, 128))
    leaf_s = _pad_reshape(leaf, NS * LPT, N, (NS, NCH_L, 128))
    lorder = jnp.argsort(leaf).astype(jnp.int32)
    lorder_s = _pad_reshape(lorder, NS * LPT, 0, (NS, NCH_L, 128))
    leafsort_s = _pad_reshape(leaf[lorder], NS * LPT, N, (NS, NCH_L, 128))

    order = jnp.argsort(dst)
    src_s = src[order]
    base_s = tuple_edge_base[order]
    dst_s = dst[order]
    bounds = jnp.searchsorted(dst_s, jnp.arange(0, T + 1, RP, dtype=jnp.int32))
    start_c = (bounds[:-1] // 128).astype(jnp.int32)
    end_c = ((bounds[1:] + 127) // 128).astype(jnp.int32)
    meta = jnp.concatenate([start_c, end_c - start_c]).astype(jnp.int32)

    b_ti = b_tupleinit.reshape(1, D)
    b_m = b_merge.reshape(1, D)
    b_p = b_pred.reshape(1, NUM_TASKS)

    xt = tc_xt(x, W_tupleinit, b_ti)
    G = sc_gather_rows(xt, leaf_g)
    X, X2 = tc_init(x, G, tuplefeat)
    ea2 = tc_split(edge_attr)
    for l in range(conv_W.shape[0]):
        agg = sc_layer(X2, ea2, src_s, base_s, dst_s, meta)
        X, X2 = tc_conv(X, agg, conv_W[l], conv_b[l].reshape(1, D))
    x2sum = sc_pool_leaf(X2, lorder_s, leafsort_s)
    hist = sc_hist(leaf_s)

    hist_r = hist[:, :N].reshape(NS, NBLK, NB).transpose(1, 0, 2)
    batch_r = batch.reshape(NBLK, 1, NB)
    out = tc_final(X, x2sum, hist_r, batch_r, W_merge, b_m, W_pred, b_p)
    return out
